# baseline (speedup 1.0000x reference)
import functools

import jax
import jax.numpy as jnp
from jax import lax
from jax.experimental import pallas as pl
from jax.experimental.pallas import tpu as pltpu
from jax.experimental.pallas import tpu_sc as plsc

N = 100000
D = 10
DP = 16
E = 6400000
NC = 2
NS = 16
NW = NC * NS
NROWS = 100352
CHUNK = 512
NCHUNK = E // CHUNK
BASE_STEPS = NCHUNK // NW
XTRA = NCHUNK - BASE_STEPS * NW
RING = 4
NSLOT = 3
M = NROWS // 8
BM = 256
GRID = M // BM


def _sc_aggregate_body(hm_hbm, edges_hbm, out_hbm,
                       acc, idxbuf, rows, isem, gsem, ssem):
    c = lax.axis_index("c")
    s = lax.axis_index("s")
    wid = c * NS + s
    rpt = NROWS // NS

    def zstore(i, carry):
        rows[0, i] = jnp.zeros((DP,), jnp.float32)
        return carry

    lax.fori_loop(0, CHUNK, zstore, 0)
    nfull = rpt // CHUNK
    rem = rpt - nfull * CHUNK
    for k in range(nfull):
        pltpu.async_copy(rows.at[0], acc.at[pl.ds(s * rpt + k * CHUNK, CHUNK)],
                         gsem)
    if rem:
        pltpu.async_copy(rows.at[0, pl.ds(0, rem)],
                         acc.at[pl.ds(s * rpt + nfull * CHUNK, rem)], gsem)
    for k in range(nfull):
        pltpu.make_async_copy(rows.at[0], acc.at[pl.ds(0, CHUNK)], gsem).wait()
    if rem:
        pltpu.make_async_copy(rows.at[0, pl.ds(0, rem)], acc.at[pl.ds(0, rem)],
                              gsem).wait()
    plsc.subcore_barrier()

    steps = BASE_STEPS + jnp.where(wid < XTRA, 1, 0)
    chunk0 = wid * BASE_STEPS + jnp.minimum(wid, XTRA)

    def fire_idx(ci, ri):
        ebase = (chunk0 + ci) * CHUNK
        pltpu.async_copy(edges_hbm.at[:, pl.ds(ebase, CHUNK)], idxbuf.at[ri],
                         isem.at[ri])

    def wait_idx(ri):
        pltpu.make_async_copy(edges_hbm.at[:, pl.ds(0, CHUNK)], idxbuf.at[ri],
                              isem.at[ri]).wait()

    def fire_gather(ci, slot):
        pltpu.async_copy(hm_hbm.at[idxbuf.at[lax.rem(ci, RING), 1]],
                         rows.at[slot], gsem)

    def wait_gather(slot):
        pltpu.make_async_copy(hm_hbm.at[idxbuf.at[0, 1]], rows.at[slot],
                              gsem).wait()

    def fire_scatter(ci, slot):
        pltpu.async_copy(rows.at[slot], acc.at[idxbuf.at[lax.rem(ci, RING), 0]],
                         ssem.at[slot], add=True)

    def drain_scatter(slot):
        pltpu.make_async_copy(rows.at[0], acc.at[idxbuf.at[0, 0]],
                              ssem.at[slot]).wait()

    def step(ci, carry):
        slot = lax.rem(ci, NSLOT)
        wait_gather(slot)
        fire_scatter(ci, slot)

        @pl.when(ci >= 2)
        def _():
            drain_scatter(lax.rem(ci + 1, NSLOT))

        @pl.when(ci + 1 < steps)
        def _():
            wait_idx(lax.rem(ci + 1, RING))
            fire_gather(ci + 1, lax.rem(ci + 1, NSLOT))

        @pl.when(ci + 2 < steps)
        def _():
            fire_idx(ci + 2, lax.rem(ci + 2, RING))

        return carry

    fire_idx(0, 0)
    fire_idx(1, 1)
    wait_idx(0)
    fire_gather(0, 0)
    lax.fori_loop(0, steps, step, 0)
    drain_scatter(lax.rem(steps - 1, NSLOT))
    drain_scatter(lax.rem(steps - 2, NSLOT))
    plsc.subcore_barrier()
    pltpu.sync_copy(acc.at[pl.ds(s * rpt, rpt)], out_hbm.at[c, pl.ds(s * rpt, rpt)])


@jax.jit
def _sc_aggregate(hm, edges):
    mesh = plsc.VectorSubcoreMesh(core_axis_name="c", subcore_axis_name="s")
    return pl.kernel(
        _sc_aggregate_body,
        out_type=jax.ShapeDtypeStruct((NC, NROWS, DP), jnp.float32),
        mesh=mesh,
        scratch_types=[
            pltpu.VMEM_SHARED((NROWS, DP), jnp.float32),
            pltpu.VMEM((RING, 2, CHUNK), jnp.int32),
            pltpu.VMEM((NSLOT, CHUNK, DP), jnp.float32),
            pltpu.SemaphoreType.DMA((RING,)),
            pltpu.SemaphoreType.DMA,
            pltpu.SemaphoreType.DMA((NSLOT,)),
        ],
        compiler_params=pltpu.CompilerParams(use_tc_tiling_on_sc=False),
    )(hm, edges)


def _gru_math(x, h, W_ref, B_ref):
    dot = functools.partial(jnp.dot, preferred_element_type=jnp.float32,
                            precision=lax.Precision.HIGHEST)
    z = jax.nn.sigmoid(dot(x, W_ref[0]) + dot(h, W_ref[1]) + B_ref[0:1, :])
    r = jax.nn.sigmoid(dot(x, W_ref[2]) + dot(h, W_ref[3]) + B_ref[1:2, :])
    hh = jnp.tanh(dot(x, W_ref[4]) + dot(r * h, W_ref[5]) + B_ref[2:3, :])
    return z * h + (1.0 - z) * hh


def _tc_gru0_body(xs_ref, h_ref, act_ref, W_ref, B_ref, hout_ref, hm1_ref):
    x = xs_ref[0] + xs_ref[1]
    h = h_ref[...]
    hn = _gru_math(x, h, W_ref, B_ref)
    hout_ref[...] = hn
    hm1_ref[...] = hn * act_ref[...]


def _tc_gru1_body(xs_ref, h_ref, act_ref, W_ref, B_ref, hout_ref):
    a = act_ref[...]
    x = (xs_ref[0] + xs_ref[1]) * a
    h = h_ref[...]
    hn = _gru_math(x, h, W_ref, B_ref)
    hout_ref[...] = jnp.where(a > 0.0, hn, h)


def _tc_specs():
    blk = pl.BlockSpec((BM, 128), lambda i: (i, 0))
    return [
        pl.BlockSpec((2, BM, 128), lambda i: (0, i, 0)),
        blk,
        blk,
        pl.BlockSpec((6, 128, 128), lambda i: (0, 0, 0)),
        pl.BlockSpec((8, 128), lambda i: (0, 0)),
    ], blk


@jax.jit
def _tc_gru0(xs, h, actp, Wbd, Bt):
    specs, blk = _tc_specs()
    out = jax.ShapeDtypeStruct((M, 128), jnp.float32)
    return pl.pallas_call(
        _tc_gru0_body,
        grid=(GRID,),
        in_specs=specs,
        out_specs=[blk, blk],
        out_shape=[out, out],
    )(xs, h, actp, Wbd, Bt)


@jax.jit
def _tc_gru1(xs, h, actp, Wbd, Bt):
    specs, blk = _tc_specs()
    out = jax.ShapeDtypeStruct((M, 128), jnp.float32)
    return pl.pallas_call(
        _tc_gru1_body,
        grid=(GRID,),
        in_specs=specs,
        out_specs=blk,
        out_shape=out,
    )(xs, h, actp, Wbd, Bt)


def _pad_w(w):
    w16 = jnp.zeros((DP, DP), jnp.float32).at[:D, :D].set(w.T)
    return jnp.kron(jnp.eye(8, dtype=jnp.float32), w16)


def _pad_b(b):
    return jnp.tile(jnp.zeros((DP,), jnp.float32).at[:D].set(b), 8)


def kernel(h, edge_index, node2depth,
           Wz_w, Wz_b, Uz_w, Uz_b,
           Wr_w, Wr_b, Ur_w, Ur_b,
           Wh_w, Wh_b, Uh_w, Uh_b):
    zeros = jnp.zeros((NROWS, DP), jnp.float32)
    h0 = zeros.at[:N, :D].set(h)
    act1 = jnp.zeros((NROWS,), jnp.float32).at[:N].set(
        (node2depth <= 1).astype(jnp.float32))
    actp = jnp.broadcast_to(act1[:, None], (NROWS, DP)).reshape(M, 128)

    Wbd = jnp.stack([_pad_w(Wz_w), _pad_w(Uz_w), _pad_w(Wr_w),
                     _pad_w(Ur_w), _pad_w(Wh_w), _pad_w(Uh_w)])
    Bt = jnp.zeros((8, 128), jnp.float32)
    Bt = Bt.at[0].set(_pad_b(Wz_b + Uz_b))
    Bt = Bt.at[1].set(_pad_b(Wr_b + Ur_b))
    Bt = Bt.at[2].set(_pad_b(Wh_b + Uh_b))

    xs0 = _sc_aggregate(h0, edge_index)
    h1, hm1 = _tc_gru0(xs0.reshape(NC, M, 128), h0.reshape(M, 128), actp, Wbd, Bt)
    xs1 = _sc_aggregate(hm1.reshape(NROWS, DP), edge_index)
    h2 = _tc_gru1(xs1.reshape(NC, M, 128), h1, actp, Wbd, Bt)
    return h2.reshape(NROWS, DP)[:N, :D]


# speedup vs baseline: 1.0760x; 1.0760x over previous
import functools

import jax
import jax.numpy as jnp
from jax import lax
from jax.experimental import pallas as pl
from jax.experimental.pallas import tpu as pltpu
from jax.experimental.pallas import tpu_sc as plsc

N = 100000
D = 10
DP = 16
E = 6400000
NC = 2
NS = 16
NW = NC * NS
NROWS = 100352
CHUNK = 640
NCHUNK = E // CHUNK
BASE_STEPS = NCHUNK // NW
XTRA = NCHUNK - BASE_STEPS * NW
RING = 4
NSLOT = 2
M = NROWS // 8
BM = 256
GRID = M // BM


def _sc_aggregate_body(hm_hbm, edges_hbm, out_hbm,
                       acc, idxbuf, rows, isem, gsem, ssem):
    c = lax.axis_index("c")
    s = lax.axis_index("s")
    wid = c * NS + s
    rpt = NROWS // NS

    def zstore(i, carry):
        rows[0, i] = jnp.zeros((DP,), jnp.float32)
        return carry

    lax.fori_loop(0, CHUNK, zstore, 0)
    nfull = rpt // CHUNK
    rem = rpt - nfull * CHUNK
    for k in range(nfull):
        pltpu.async_copy(rows.at[0], acc.at[pl.ds(s * rpt + k * CHUNK, CHUNK)],
                         gsem)
    if rem:
        pltpu.async_copy(rows.at[0, pl.ds(0, rem)],
                         acc.at[pl.ds(s * rpt + nfull * CHUNK, rem)], gsem)
    for k in range(nfull):
        pltpu.make_async_copy(rows.at[0], acc.at[pl.ds(0, CHUNK)], gsem).wait()
    if rem:
        pltpu.make_async_copy(rows.at[0, pl.ds(0, rem)], acc.at[pl.ds(0, rem)],
                              gsem).wait()
    plsc.subcore_barrier()

    steps = BASE_STEPS + jnp.where(wid < XTRA, 1, 0)
    chunk0 = wid * BASE_STEPS + jnp.minimum(wid, XTRA)

    def fire_idx(ci, ri):
        ebase = (chunk0 + ci) * CHUNK
        pltpu.async_copy(edges_hbm.at[:, pl.ds(ebase, CHUNK)], idxbuf.at[ri],
                         isem.at[ri])

    def wait_idx(ri):
        pltpu.make_async_copy(edges_hbm.at[:, pl.ds(0, CHUNK)], idxbuf.at[ri],
                              isem.at[ri]).wait()

    def fire_gather(ci, slot):
        pltpu.async_copy(hm_hbm.at[idxbuf.at[lax.rem(ci, RING), 1]],
                         rows.at[slot], gsem)

    def wait_gather(slot):
        pltpu.make_async_copy(hm_hbm.at[idxbuf.at[0, 1]], rows.at[slot],
                              gsem).wait()

    def fire_scatter(ci, slot):
        pltpu.async_copy(rows.at[slot], acc.at[idxbuf.at[lax.rem(ci, RING), 0]],
                         ssem.at[slot], add=True)

    def drain_scatter(slot):
        pltpu.make_async_copy(rows.at[0], acc.at[idxbuf.at[0, 0]],
                              ssem.at[slot]).wait()

    def step(ci, carry):
        slot = lax.rem(ci, NSLOT)
        other = lax.rem(ci + 1, NSLOT)
        wait_gather(slot)
        fire_scatter(ci, slot)

        @pl.when(ci >= 1)
        def _():
            drain_scatter(other)

        @pl.when(ci + 1 < steps)
        def _():
            wait_idx(lax.rem(ci + 1, RING))
            fire_gather(ci + 1, other)

        @pl.when(ci + 2 < steps)
        def _():
            fire_idx(ci + 2, lax.rem(ci + 2, RING))

        return carry

    fire_idx(0, 0)
    fire_idx(1, 1)
    wait_idx(0)
    fire_gather(0, 0)
    lax.fori_loop(0, steps, step, 0)
    drain_scatter(lax.rem(steps - 1, NSLOT))
    plsc.subcore_barrier()
    pltpu.sync_copy(acc.at[pl.ds(s * rpt, rpt)], out_hbm.at[c, pl.ds(s * rpt, rpt)])


@jax.jit
def _sc_aggregate(hm, edges):
    mesh = plsc.VectorSubcoreMesh(core_axis_name="c", subcore_axis_name="s")
    return pl.kernel(
        _sc_aggregate_body,
        out_type=jax.ShapeDtypeStruct((NC, NROWS, DP), jnp.float32),
        mesh=mesh,
        scratch_types=[
            pltpu.VMEM_SHARED((NROWS, DP), jnp.float32),
            pltpu.VMEM((RING, 2, CHUNK), jnp.int32),
            pltpu.VMEM((NSLOT, CHUNK, DP), jnp.float32),
            pltpu.SemaphoreType.DMA((RING,)),
            pltpu.SemaphoreType.DMA,
            pltpu.SemaphoreType.DMA((NSLOT,)),
        ],
        compiler_params=pltpu.CompilerParams(use_tc_tiling_on_sc=False),
    )(hm, edges)


def _gru_math(x, h, W_ref, B_ref):
    dot = functools.partial(jnp.dot, preferred_element_type=jnp.float32,
                            precision=lax.Precision.HIGHEST)
    z = jax.nn.sigmoid(dot(x, W_ref[0]) + dot(h, W_ref[1]) + B_ref[0:1, :])
    r = jax.nn.sigmoid(dot(x, W_ref[2]) + dot(h, W_ref[3]) + B_ref[1:2, :])
    hh = jnp.tanh(dot(x, W_ref[4]) + dot(r * h, W_ref[5]) + B_ref[2:3, :])
    return z * h + (1.0 - z) * hh


def _tc_gru0_body(xs_ref, h_ref, act_ref, W_ref, B_ref, hout_ref, hm1_ref):
    x = xs_ref[0] + xs_ref[1]
    h = h_ref[...]
    hn = _gru_math(x, h, W_ref, B_ref)
    hout_ref[...] = hn
    hm1_ref[...] = hn * act_ref[...]


def _tc_gru1_body(xs_ref, h_ref, act_ref, W_ref, B_ref, hout_ref):
    a = act_ref[...]
    x = (xs_ref[0] + xs_ref[1]) * a
    h = h_ref[...]
    hn = _gru_math(x, h, W_ref, B_ref)
    hout_ref[...] = jnp.where(a > 0.0, hn, h)


def _tc_specs():
    blk = pl.BlockSpec((BM, 128), lambda i: (i, 0))
    return [
        pl.BlockSpec((2, BM, 128), lambda i: (0, i, 0)),
        blk,
        blk,
        pl.BlockSpec((6, 128, 128), lambda i: (0, 0, 0)),
        pl.BlockSpec((8, 128), lambda i: (0, 0)),
    ], blk


@jax.jit
def _tc_gru0(xs, h, actp, Wbd, Bt):
    specs, blk = _tc_specs()
    out = jax.ShapeDtypeStruct((M, 128), jnp.float32)
    return pl.pallas_call(
        _tc_gru0_body,
        grid=(GRID,),
        in_specs=specs,
        out_specs=[blk, blk],
        out_shape=[out, out],
    )(xs, h, actp, Wbd, Bt)


@jax.jit
def _tc_gru1(xs, h, actp, Wbd, Bt):
    specs, blk = _tc_specs()
    out = jax.ShapeDtypeStruct((M, 128), jnp.float32)
    return pl.pallas_call(
        _tc_gru1_body,
        grid=(GRID,),
        in_specs=specs,
        out_specs=blk,
        out_shape=out,
    )(xs, h, actp, Wbd, Bt)


def _pad_w(w):
    w16 = jnp.zeros((DP, DP), jnp.float32).at[:D, :D].set(w.T)
    return jnp.kron(jnp.eye(8, dtype=jnp.float32), w16)


def _pad_b(b):
    return jnp.tile(jnp.zeros((DP,), jnp.float32).at[:D].set(b), 8)


def kernel(h, edge_index, node2depth,
           Wz_w, Wz_b, Uz_w, Uz_b,
           Wr_w, Wr_b, Ur_w, Ur_b,
           Wh_w, Wh_b, Uh_w, Uh_b):
    zeros = jnp.zeros((NROWS, DP), jnp.float32)
    h0 = zeros.at[:N, :D].set(h)
    act1 = jnp.zeros((NROWS,), jnp.float32).at[:N].set(
        (node2depth <= 1).astype(jnp.float32))
    actp = jnp.broadcast_to(act1[:, None], (NROWS, DP)).reshape(M, 128)

    Wbd = jnp.stack([_pad_w(Wz_w), _pad_w(Uz_w), _pad_w(Wr_w),
                     _pad_w(Ur_w), _pad_w(Wh_w), _pad_w(Uh_w)])
    Bt = jnp.zeros((8, 128), jnp.float32)
    Bt = Bt.at[0].set(_pad_b(Wz_b + Uz_b))
    Bt = Bt.at[1].set(_pad_b(Wr_b + Ur_b))
    Bt = Bt.at[2].set(_pad_b(Wh_b + Uh_b))

    xs0 = _sc_aggregate(h0, edge_index)
    h1, hm1 = _tc_gru0(xs0.reshape(NC, M, 128), h0.reshape(M, 128), actp, Wbd, Bt)
    xs1 = _sc_aggregate(hm1.reshape(NROWS, DP), edge_index)
    h2 = _tc_gru1(xs1.reshape(NC, M, 128), h1, actp, Wbd, Bt)
    return h2.reshape(NROWS, DP)[:N, :D]


# speedup vs baseline: 1.1223x; 1.0431x over previous
import functools

import jax
import jax.numpy as jnp
from jax import lax
from jax.experimental import pallas as pl
from jax.experimental.pallas import tpu as pltpu
from jax.experimental.pallas import tpu_sc as plsc

N = 100000
D = 10
DP = 16
E = 6400000
NC = 2
NS = 16
NW = NC * NS
NROWS = 100352
CHUNK = 640
NCHUNK = E // CHUNK
BASE_STEPS = NCHUNK // NW
XTRA = NCHUNK - BASE_STEPS * NW
RING = 4
NSLOT = 2
M = NROWS // 8
BM = 256
GRID = M // BM


def _sc_aggregate_body(hm_hbm, edges_hbm, out_hbm,
                       acc, idxbuf, rows, isem, gsem, ssem):
    c = lax.axis_index("c")
    s = lax.axis_index("s")
    wid = c * NS + s
    rpt = NROWS // NS

    def zstore(i, carry):
        rows[0, i] = jnp.zeros((DP,), jnp.float32)
        return carry

    lax.fori_loop(0, CHUNK, zstore, 0)
    nfull = rpt // CHUNK
    rem = rpt - nfull * CHUNK
    for k in range(nfull):
        pltpu.async_copy(rows.at[0], acc.at[pl.ds(s * rpt + k * CHUNK, CHUNK)],
                         gsem)
    if rem:
        pltpu.async_copy(rows.at[0, pl.ds(0, rem)],
                         acc.at[pl.ds(s * rpt + nfull * CHUNK, rem)], gsem)
    for k in range(nfull):
        pltpu.make_async_copy(rows.at[0], acc.at[pl.ds(0, CHUNK)], gsem).wait()
    if rem:
        pltpu.make_async_copy(rows.at[0, pl.ds(0, rem)], acc.at[pl.ds(0, rem)],
                              gsem).wait()
    plsc.subcore_barrier()

    steps = BASE_STEPS + jnp.where(wid < XTRA, 1, 0)
    chunk0 = wid * BASE_STEPS + jnp.minimum(wid, XTRA)

    def fire_idx(ci, ri):
        ebase = (chunk0 + ci) * CHUNK
        pltpu.async_copy(edges_hbm.at[:, pl.ds(ebase, CHUNK)], idxbuf.at[ri],
                         isem.at[ri])

    def wait_idx(ri):
        pltpu.make_async_copy(edges_hbm.at[:, pl.ds(0, CHUNK)], idxbuf.at[ri],
                              isem.at[ri]).wait()

    def fire_gather(ci, slot):
        pltpu.async_copy(hm_hbm.at[idxbuf.at[lax.rem(ci, RING), 1]],
                         rows.at[slot], gsem)

    def wait_gather(slot):
        pltpu.make_async_copy(hm_hbm.at[idxbuf.at[0, 1]], rows.at[slot],
                              gsem).wait()

    def fire_scatter(ci, slot):
        pltpu.async_copy(rows.at[slot], acc.at[idxbuf.at[lax.rem(ci, RING), 0]],
                         ssem.at[slot], add=True)

    def drain_scatter(slot):
        pltpu.make_async_copy(rows.at[0], acc.at[idxbuf.at[0, 0]],
                              ssem.at[slot]).wait()

    def step(ci, carry):
        slot = lax.rem(ci, NSLOT)
        other = lax.rem(ci + 1, NSLOT)
        wait_gather(slot)
        fire_scatter(ci, slot)

        @pl.when(ci >= 1)
        def _():
            drain_scatter(other)

        @pl.when(ci + 1 < steps)
        def _():
            wait_idx(lax.rem(ci + 1, RING))
            fire_gather(ci + 1, other)

        @pl.when(ci + 2 < steps)
        def _():
            fire_idx(ci + 2, lax.rem(ci + 2, RING))

        return carry

    fire_idx(0, 0)
    fire_idx(1, 1)
    wait_idx(0)
    fire_gather(0, 0)
    lax.fori_loop(0, steps, step, 0)
    drain_scatter(lax.rem(steps - 1, NSLOT))
    plsc.subcore_barrier()
    pltpu.sync_copy(acc.at[pl.ds(s * rpt, rpt)], out_hbm.at[c, pl.ds(s * rpt, rpt)])


@jax.jit
def _sc_aggregate(hm, edges):
    mesh = plsc.VectorSubcoreMesh(core_axis_name="c", subcore_axis_name="s")
    return pl.kernel(
        _sc_aggregate_body,
        out_type=jax.ShapeDtypeStruct((NC, NROWS, DP), jnp.float32),
        mesh=mesh,
        scratch_types=[
            pltpu.VMEM_SHARED((NROWS, DP), jnp.float32),
            pltpu.VMEM((RING, 2, CHUNK), jnp.int32),
            pltpu.VMEM((NSLOT, CHUNK, DP), jnp.float32),
            pltpu.SemaphoreType.DMA((RING,)),
            pltpu.SemaphoreType.DMA,
            pltpu.SemaphoreType.DMA((NSLOT,)),
        ],
        compiler_params=pltpu.CompilerParams(use_tc_tiling_on_sc=False),
    )(hm, edges)


def _gru_math(x, h, W_ref, B_ref):
    dot = functools.partial(jnp.dot, preferred_element_type=jnp.float32)
    z = jax.nn.sigmoid(dot(x, W_ref[0]) + dot(h, W_ref[1]) + B_ref[0:1, :])
    r = jax.nn.sigmoid(dot(x, W_ref[2]) + dot(h, W_ref[3]) + B_ref[1:2, :])
    hh = jnp.tanh(dot(x, W_ref[4]) + dot(r * h, W_ref[5]) + B_ref[2:3, :])
    return z * h + (1.0 - z) * hh


def _tc_gru0_body(xs_ref, h_ref, act_ref, W_ref, B_ref, hout_ref, hm1_ref):
    x = xs_ref[0] + xs_ref[1]
    h = h_ref[...]
    hn = _gru_math(x, h, W_ref, B_ref)
    hout_ref[...] = hn
    hm1_ref[...] = hn * act_ref[...]


def _tc_gru1_body(xs_ref, h_ref, act_ref, W_ref, B_ref, hout_ref):
    a = act_ref[...]
    x = (xs_ref[0] + xs_ref[1]) * a
    h = h_ref[...]
    hn = _gru_math(x, h, W_ref, B_ref)
    hout_ref[...] = jnp.where(a > 0.0, hn, h)


def _tc_specs():
    blk = pl.BlockSpec((BM, 128), lambda i: (i, 0))
    return [
        pl.BlockSpec((2, BM, 128), lambda i: (0, i, 0)),
        blk,
        blk,
        pl.BlockSpec((6, 128, 128), lambda i: (0, 0, 0)),
        pl.BlockSpec((8, 128), lambda i: (0, 0)),
    ], blk


@jax.jit
def _tc_gru0(xs, h, actp, Wbd, Bt):
    specs, blk = _tc_specs()
    out = jax.ShapeDtypeStruct((M, 128), jnp.float32)
    return pl.pallas_call(
        _tc_gru0_body,
        grid=(GRID,),
        in_specs=specs,
        out_specs=[blk, blk],
        out_shape=[out, out],
    )(xs, h, actp, Wbd, Bt)


@jax.jit
def _tc_gru1(xs, h, actp, Wbd, Bt):
    specs, blk = _tc_specs()
    return pl.pallas_call(
        _tc_gru1_body,
        grid=(GRID,),
        in_specs=specs,
        out_specs=blk,
        out_shape=jax.ShapeDtypeStruct((M, 128), jnp.float32),
    )(xs, h, actp, Wbd, Bt)


def _pad_w(w):
    w16 = jnp.zeros((DP, DP), jnp.float32).at[:D, :D].set(w.T)
    return jnp.kron(jnp.eye(8, dtype=jnp.float32), w16)


def _pad_b(b):
    return jnp.tile(jnp.zeros((DP,), jnp.float32).at[:D].set(b), 8)


def kernel(h, edge_index, node2depth,
           Wz_w, Wz_b, Uz_w, Uz_b,
           Wr_w, Wr_b, Ur_w, Ur_b,
           Wh_w, Wh_b, Uh_w, Uh_b):
    h0 = lax.pad(h, jnp.float32(0), ((0, NROWS - N, 0), (0, DP - D, 0)))
    act1 = jnp.zeros((NROWS,), jnp.float32).at[:N].set(
        (node2depth <= 1).astype(jnp.float32))
    actp = jnp.broadcast_to(act1[:, None], (NROWS, DP)).reshape(M, 128)

    Wbd = jnp.stack([_pad_w(Wz_w), _pad_w(Uz_w), _pad_w(Wr_w),
                     _pad_w(Ur_w), _pad_w(Wh_w), _pad_w(Uh_w)])
    Bt = jnp.zeros((8, 128), jnp.float32)
    Bt = Bt.at[0].set(_pad_b(Wz_b + Uz_b))
    Bt = Bt.at[1].set(_pad_b(Wr_b + Ur_b))
    Bt = Bt.at[2].set(_pad_b(Wh_b + Uh_b))

    xs0 = _sc_aggregate(h0, edge_index)
    h1, hm1 = _tc_gru0(xs0.reshape(NC, M, 128), h0.reshape(M, 128), actp, Wbd, Bt)
    xs1 = _sc_aggregate(hm1.reshape(NROWS, DP), edge_index)
    h2 = _tc_gru1(xs1.reshape(NC, M, 128), h1, actp, Wbd, Bt)
    return h2.reshape(NROWS, DP)[:N, :D]


# speedup vs baseline: 1.1629x; 1.0361x over previous
import functools

import jax
import jax.numpy as jnp
from jax import lax
from jax.experimental import pallas as pl
from jax.experimental.pallas import tpu as pltpu
from jax.experimental.pallas import tpu_sc as plsc

N = 100000
D = 10
DP = 16
E = 6400000
NC = 2
NS = 16
NW = NC * NS
NROWS = 100352
CHUNK = 640
NCHUNK = E // CHUNK
BASE_STEPS = NCHUNK // NW
XTRA = NCHUNK - BASE_STEPS * NW
RING = 4
NSLOT = 2
M = NROWS // 8
BM = 256
GRID = M // BM


def _sc_aggregate_body(hm_hbm, edges_hbm, out_hbm,
                       acc, idxbuf, rows, isem, gsem, ssem):
    c = lax.axis_index("c")
    s = lax.axis_index("s")
    wid = c * NS + s
    rpt = NROWS // NS

    def zstore(i, carry):
        rows[0, i] = jnp.zeros((DP,), jnp.float32)
        return carry

    lax.fori_loop(0, CHUNK, zstore, 0)
    nfull = rpt // CHUNK
    rem = rpt - nfull * CHUNK
    for k in range(nfull):
        pltpu.async_copy(rows.at[0], acc.at[pl.ds(s * rpt + k * CHUNK, CHUNK)],
                         gsem)
    if rem:
        pltpu.async_copy(rows.at[0, pl.ds(0, rem)],
                         acc.at[pl.ds(s * rpt + nfull * CHUNK, rem)], gsem)
    for k in range(nfull):
        pltpu.make_async_copy(rows.at[0], acc.at[pl.ds(0, CHUNK)], gsem).wait()
    if rem:
        pltpu.make_async_copy(rows.at[0, pl.ds(0, rem)], acc.at[pl.ds(0, rem)],
                              gsem).wait()
    plsc.subcore_barrier()

    steps = BASE_STEPS + jnp.where(wid < XTRA, 1, 0)
    chunk0 = wid * BASE_STEPS + jnp.minimum(wid, XTRA)

    def fire_idx(ci, ri):
        ebase = (chunk0 + ci) * CHUNK
        pltpu.async_copy(edges_hbm.at[:, pl.ds(ebase, CHUNK)], idxbuf.at[ri],
                         isem.at[ri])

    def wait_idx(ri):
        pltpu.make_async_copy(edges_hbm.at[:, pl.ds(0, CHUNK)], idxbuf.at[ri],
                              isem.at[ri]).wait()

    def fire_gather(ci, slot):
        pltpu.async_copy(hm_hbm.at[idxbuf.at[lax.rem(ci, RING), 1]],
                         rows.at[slot], gsem)

    def wait_gather(slot):
        pltpu.make_async_copy(hm_hbm.at[idxbuf.at[0, 1]], rows.at[slot],
                              gsem).wait()

    def fire_scatter(ci, slot):
        pltpu.async_copy(rows.at[slot], acc.at[idxbuf.at[lax.rem(ci, RING), 0]],
                         ssem.at[slot], add=True)

    def drain_scatter(slot):
        pltpu.make_async_copy(rows.at[0], acc.at[idxbuf.at[0, 0]],
                              ssem.at[slot]).wait()

    def step(ci, carry):
        slot = lax.rem(ci, NSLOT)
        other = lax.rem(ci + 1, NSLOT)
        wait_gather(slot)
        fire_scatter(ci, slot)

        @pl.when(ci >= 1)
        def _():
            drain_scatter(other)

        @pl.when(ci + 1 < steps)
        def _():
            wait_idx(lax.rem(ci + 1, RING))
            fire_gather(ci + 1, other)

        @pl.when(ci + 2 < steps)
        def _():
            fire_idx(ci + 2, lax.rem(ci + 2, RING))

        return carry

    fire_idx(0, 0)
    fire_idx(1, 1)
    wait_idx(0)
    fire_gather(0, 0)
    lax.fori_loop(0, steps, step, 0)
    drain_scatter(lax.rem(steps - 1, NSLOT))
    plsc.subcore_barrier()
    pltpu.sync_copy(acc.at[pl.ds(s * rpt, rpt)], out_hbm.at[c, pl.ds(s * rpt, rpt)])


@jax.jit
def _sc_aggregate(hm, edges):
    mesh = plsc.VectorSubcoreMesh(core_axis_name="c", subcore_axis_name="s")
    return pl.kernel(
        _sc_aggregate_body,
        out_type=jax.ShapeDtypeStruct((NC, NROWS, DP), jnp.float32),
        mesh=mesh,
        scratch_types=[
            pltpu.VMEM_SHARED((NROWS, DP), jnp.float32),
            pltpu.VMEM((RING, 2, CHUNK), jnp.int32),
            pltpu.VMEM((NSLOT, CHUNK, DP), jnp.float32),
            pltpu.SemaphoreType.DMA((RING,)),
            pltpu.SemaphoreType.DMA,
            pltpu.SemaphoreType.DMA((NSLOT,)),
        ],
        compiler_params=pltpu.CompilerParams(use_tc_tiling_on_sc=False),
    )(hm, edges)


def _gru_math(x, h, W_ref, B_ref):
    dot = functools.partial(jnp.dot, preferred_element_type=jnp.float32)
    z = jax.nn.sigmoid(dot(x, W_ref[0]) + dot(h, W_ref[1]) + B_ref[0:1, :])
    r = jax.nn.sigmoid(dot(x, W_ref[2]) + dot(h, W_ref[3]) + B_ref[1:2, :])
    hh = jnp.tanh(dot(x, W_ref[4]) + dot(r * h, W_ref[5]) + B_ref[2:3, :])
    return z * h + (1.0 - z) * hh


def _tc_gru0_body(xs_ref, h_ref, act_ref, W_ref, B_ref, hout_ref, hm1_ref):
    x = xs_ref[0] + xs_ref[1]
    h = h_ref[...]
    hn = _gru_math(x, h, W_ref, B_ref)
    hout_ref[...] = hn
    hm1_ref[...] = hn * act_ref[...]


def _tc_gru1_body(xs_ref, h_ref, act_ref, W_ref, B_ref, S_ref, hout_ref):
    a = act_ref[...]
    x = xs_ref[0] + xs_ref[1]
    h = h_ref[...]
    hn = _gru_math(x, h, W_ref, B_ref)
    hout_ref[...] = jnp.dot(jnp.where(a > 0.0, hn, h), S_ref[...],
                            preferred_element_type=jnp.float32,
                            precision=lax.Precision.HIGHEST)


def _tc_specs():
    blk = pl.BlockSpec((BM, 128), lambda i: (i, 0))
    return [
        pl.BlockSpec((2, BM, 128), lambda i: (0, i, 0)),
        blk,
        blk,
        pl.BlockSpec((6, 128, 128), lambda i: (0, 0, 0)),
        pl.BlockSpec((8, 128), lambda i: (0, 0)),
    ], blk


@jax.jit
def _tc_gru0(xs, h, actp, Wbd, Bt):
    specs, blk = _tc_specs()
    out = jax.ShapeDtypeStruct((M, 128), jnp.float32)
    return pl.pallas_call(
        _tc_gru0_body,
        grid=(GRID,),
        in_specs=specs,
        out_specs=[blk, blk],
        out_shape=[out, out],
    )(xs, h, actp, Wbd, Bt)


@jax.jit
def _tc_gru1(xs, h, actp, Wbd, Bt, S):
    specs, blk = _tc_specs()
    return pl.pallas_call(
        _tc_gru1_body,
        grid=(GRID,),
        in_specs=specs + [pl.BlockSpec((128, 80), lambda i: (0, 0))],
        out_specs=pl.BlockSpec((BM, 80), lambda i: (i, 0)),
        out_shape=jax.ShapeDtypeStruct((M, 80), jnp.float32),
    )(xs, h, actp, Wbd, Bt, S)


def _pad_w(w):
    w16 = jnp.zeros((DP, DP), jnp.float32).at[:D, :D].set(w.T)
    return jnp.kron(jnp.eye(8, dtype=jnp.float32), w16)


def _pad_b(b):
    return jnp.tile(jnp.zeros((DP,), jnp.float32).at[:D].set(b), 8)


def kernel(h, edge_index, node2depth,
           Wz_w, Wz_b, Uz_w, Uz_b,
           Wr_w, Wr_b, Ur_w, Ur_b,
           Wh_w, Wh_b, Uh_w, Uh_b):
    kk = jnp.arange(8, dtype=jnp.int32)[:, None]
    dd = jnp.arange(D, dtype=jnp.int32)[None, :]
    sel_r = (DP * kk + dd).reshape(-1)
    S_in = jnp.zeros((8 * D, 128), jnp.float32).at[
        jnp.arange(8 * D), sel_r].set(1.0)
    S_out = S_in.T

    h0c = h.reshape(N // 8, 8 * D)
    h0 = jnp.pad(jnp.dot(h0c, S_in, preferred_element_type=jnp.float32,
                         precision=lax.Precision.HIGHEST),
                 ((0, M - N // 8), (0, 0))).reshape(NROWS, DP)
    act1 = jnp.zeros((NROWS,), jnp.float32).at[:N].set(
        (node2depth <= 1).astype(jnp.float32))
    actp = jnp.broadcast_to(act1[:, None], (NROWS, DP)).reshape(M, 128)

    Wbd = jnp.stack([_pad_w(Wz_w), _pad_w(Uz_w), _pad_w(Wr_w),
                     _pad_w(Ur_w), _pad_w(Wh_w), _pad_w(Uh_w)])
    Bt = jnp.zeros((8, 128), jnp.float32)
    Bt = Bt.at[0].set(_pad_b(Wz_b + Uz_b))
    Bt = Bt.at[1].set(_pad_b(Wr_b + Ur_b))
    Bt = Bt.at[2].set(_pad_b(Wh_b + Uh_b))

    xs0 = _sc_aggregate(h0, edge_index)
    h1, hm1 = _tc_gru0(xs0.reshape(NC, M, 128), h0.reshape(M, 128), actp, Wbd, Bt)
    xs1 = _sc_aggregate(hm1.reshape(NROWS, DP), edge_index)
    h2 = _tc_gru1(xs1.reshape(NC, M, 128), h1, actp, Wbd, Bt, S_out)
    return h2[:N // 8].reshape(N, D)


# speedup vs baseline: 1.1638x; 1.0008x over previous
import functools

import jax
import jax.numpy as jnp
from jax import lax
from jax.experimental import pallas as pl
from jax.experimental.pallas import tpu as pltpu
from jax.experimental.pallas import tpu_sc as plsc

N = 100000
D = 10
DP = 16
E = 6400000
NC = 2
NS = 16
NW = NC * NS
NROWS = 100352
CHUNK = 640
NCHUNK = E // CHUNK
BASE_STEPS = NCHUNK // NW
XTRA = NCHUNK - BASE_STEPS * NW
RING = 4
NSLOT = 2
M = NROWS // 8
BM = 256
GRID = M // BM


def _sc_aggregate_body(hm_hbm, edges_hbm, out_hbm,
                       acc, idxbuf, rows, isem, gsem, ssem):
    c = lax.axis_index("c")
    s = lax.axis_index("s")
    wid = c * NS + s
    rpt = NROWS // NS

    def zstore(i, carry):
        for sl in range(NSLOT):
            rows[sl, i] = jnp.zeros((DP,), jnp.float32)
        return carry

    lax.fori_loop(0, CHUNK, zstore, 0)
    nfull = rpt // CHUNK
    rem = rpt - nfull * CHUNK
    for k in range(nfull):
        pltpu.async_copy(rows.at[0], acc.at[pl.ds(s * rpt + k * CHUNK, CHUNK)],
                         gsem)
    if rem:
        pltpu.async_copy(rows.at[0, pl.ds(0, rem)],
                         acc.at[pl.ds(s * rpt + nfull * CHUNK, rem)], gsem)
    for k in range(nfull):
        pltpu.make_async_copy(rows.at[0], acc.at[pl.ds(0, CHUNK)], gsem).wait()
    if rem:
        pltpu.make_async_copy(rows.at[0, pl.ds(0, rem)], acc.at[pl.ds(0, rem)],
                              gsem).wait()
    plsc.subcore_barrier()

    steps = BASE_STEPS + jnp.where(wid < XTRA, 1, 0)
    chunk0 = wid * BASE_STEPS + jnp.minimum(wid, XTRA)

    def fire_idx(ci, ri):
        ebase = (chunk0 + ci) * CHUNK
        pltpu.async_copy(edges_hbm.at[:, pl.ds(ebase, CHUNK)], idxbuf.at[ri],
                         isem.at[ri])

    def wait_idx(ri):
        pltpu.make_async_copy(edges_hbm.at[:, pl.ds(0, CHUNK)], idxbuf.at[ri],
                              isem.at[ri]).wait()

    def fire_gather(ci, slot):
        pltpu.async_copy(hm_hbm.at[idxbuf.at[lax.rem(ci, RING), 1]],
                         rows.at[slot], gsem)

    def wait_gather(slot):
        pltpu.make_async_copy(hm_hbm.at[idxbuf.at[0, 1]], rows.at[slot],
                              gsem).wait()

    def fire_scatter(ci, slot):
        pltpu.async_copy(rows.at[slot], acc.at[idxbuf.at[lax.rem(ci, RING), 0]],
                         ssem.at[slot], add=True)

    def drain_scatter(slot):
        pltpu.make_async_copy(rows.at[0], acc.at[idxbuf.at[0, 0]],
                              ssem.at[slot]).wait()

    def step(ci, carry):
        slot = lax.rem(ci, NSLOT)
        other = lax.rem(ci + 1, NSLOT)
        wait_gather(slot)
        fire_scatter(ci, slot)

        @pl.when(ci >= 1)
        def _():
            drain_scatter(other)

        @pl.when(ci + 1 < steps)
        def _():
            wait_idx(lax.rem(ci + 1, RING))
            fire_gather(ci + 1, other)

        @pl.when(ci + 2 < steps)
        def _():
            fire_idx(ci + 2, lax.rem(ci + 2, RING))

        return carry

    fire_idx(0, 0)
    fire_idx(1, 1)
    wait_idx(0)
    fire_gather(0, 0)
    lax.fori_loop(0, steps, step, 0)
    drain_scatter(lax.rem(steps - 1, NSLOT))
    plsc.subcore_barrier()
    pltpu.sync_copy(acc.at[pl.ds(s * rpt, rpt)], out_hbm.at[c, pl.ds(s * rpt, rpt)])


@jax.jit
def _sc_aggregate(hm, edges):
    mesh = plsc.VectorSubcoreMesh(core_axis_name="c", subcore_axis_name="s")
    return pl.kernel(
        _sc_aggregate_body,
        out_type=jax.ShapeDtypeStruct((NC, NROWS, DP), jnp.float32),
        mesh=mesh,
        scratch_types=[
            pltpu.VMEM_SHARED((NROWS, DP), jnp.float32),
            pltpu.VMEM((RING, 2, CHUNK), jnp.int32),
            pltpu.VMEM((NSLOT, CHUNK, DP), jnp.float32),
            pltpu.SemaphoreType.DMA((RING,)),
            pltpu.SemaphoreType.DMA,
            pltpu.SemaphoreType.DMA((NSLOT,)),
        ],
        compiler_params=pltpu.CompilerParams(use_tc_tiling_on_sc=False),
    )(hm, edges)


def _gru_math(x, h, W_ref, B_ref):
    dot = functools.partial(jnp.dot, preferred_element_type=jnp.float32)
    z = jax.nn.sigmoid(dot(x, W_ref[0]) + dot(h, W_ref[1]) + B_ref[0:1, :])
    r = jax.nn.sigmoid(dot(x, W_ref[2]) + dot(h, W_ref[3]) + B_ref[1:2, :])
    hh = jnp.tanh(dot(x, W_ref[4]) + dot(r * h, W_ref[5]) + B_ref[2:3, :])
    return z * h + (1.0 - z) * hh


def _tc_gru0_body(xs_ref, h_ref, act_ref, W_ref, B_ref, hout_ref, hm1_ref):
    x = xs_ref[0] + xs_ref[1]
    h = h_ref[...]
    hn = _gru_math(x, h, W_ref, B_ref)
    hout_ref[...] = hn
    hm1_ref[...] = hn * act_ref[...]


def _tc_gru1_body(xs_ref, h_ref, act_ref, W_ref, B_ref, S_ref, hout_ref):
    a = act_ref[...]
    x = xs_ref[0] + xs_ref[1]
    h = h_ref[...]
    hn = _gru_math(x, h, W_ref, B_ref)
    hout_ref[...] = jnp.dot(jnp.where(a > 0.0, hn, h), S_ref[...],
                            preferred_element_type=jnp.float32,
                            precision=lax.Precision.HIGHEST)


def _tc_specs():
    blk = pl.BlockSpec((BM, 128), lambda i: (i, 0))
    return [
        pl.BlockSpec((2, BM, 128), lambda i: (0, i, 0)),
        blk,
        blk,
        pl.BlockSpec((6, 128, 128), lambda i: (0, 0, 0)),
        pl.BlockSpec((8, 128), lambda i: (0, 0)),
    ], blk


@jax.jit
def _tc_gru0(xs, h, actp, Wbd, Bt):
    specs, blk = _tc_specs()
    out = jax.ShapeDtypeStruct((M, 128), jnp.float32)
    return pl.pallas_call(
        _tc_gru0_body,
        grid=(GRID,),
        in_specs=specs,
        out_specs=[blk, blk],
        out_shape=[out, out],
    )(xs, h, actp, Wbd, Bt)


@jax.jit
def _tc_gru1(xs, h, actp, Wbd, Bt, S):
    specs, blk = _tc_specs()
    return pl.pallas_call(
        _tc_gru1_body,
        grid=(GRID,),
        in_specs=specs + [pl.BlockSpec((128, 80), lambda i: (0, 0))],
        out_specs=pl.BlockSpec((BM, 80), lambda i: (i, 0)),
        out_shape=jax.ShapeDtypeStruct((M, 80), jnp.float32),
    )(xs, h, actp, Wbd, Bt, S)


def _pad_w(w):
    w16 = jnp.zeros((DP, DP), jnp.float32).at[:D, :D].set(w.T)
    return jnp.kron(jnp.eye(8, dtype=jnp.float32), w16)


def _pad_b(b):
    return jnp.tile(jnp.zeros((DP,), jnp.float32).at[:D].set(b), 8)


def kernel(h, edge_index, node2depth,
           Wz_w, Wz_b, Uz_w, Uz_b,
           Wr_w, Wr_b, Ur_w, Ur_b,
           Wh_w, Wh_b, Uh_w, Uh_b):
    kk = jnp.arange(8, dtype=jnp.int32)[:, None]
    dd = jnp.arange(D, dtype=jnp.int32)[None, :]
    sel_r = (DP * kk + dd).reshape(-1)
    S_in = jnp.zeros((8 * D, 128), jnp.float32).at[
        jnp.arange(8 * D), sel_r].set(1.0)
    S_out = S_in.T

    h0c = h.reshape(N // 8, 8 * D)
    h0 = jnp.pad(jnp.dot(h0c, S_in, preferred_element_type=jnp.float32,
                         precision=lax.Precision.HIGHEST),
                 ((0, M - N // 8), (0, 0))).reshape(NROWS, DP)
    act1 = jnp.zeros((NROWS,), jnp.float32).at[:N].set(
        (node2depth <= 1).astype(jnp.float32))
    actp = jnp.broadcast_to(act1[:, None], (NROWS, DP)).reshape(M, 128)

    Wbd = jnp.stack([_pad_w(Wz_w), _pad_w(Uz_w), _pad_w(Wr_w),
                     _pad_w(Ur_w), _pad_w(Wh_w), _pad_w(Uh_w)])
    Bt = jnp.zeros((8, 128), jnp.float32)
    Bt = Bt.at[0].set(_pad_b(Wz_b + Uz_b))
    Bt = Bt.at[1].set(_pad_b(Wr_b + Ur_b))
    Bt = Bt.at[2].set(_pad_b(Wh_b + Uh_b))

    xs0 = _sc_aggregate(h0, edge_index)
    h1, hm1 = _tc_gru0(xs0.reshape(NC, M, 128), h0.reshape(M, 128), actp, Wbd, Bt)
    xs1 = _sc_aggregate(hm1.reshape(NROWS, DP), edge_index)
    h2 = _tc_gru1(xs1.reshape(NC, M, 128), h1, actp, Wbd, Bt, S_out)
    return h2[:N // 8].reshape(N, D)
